# baseline (speedup 1.0000x reference)
import jax
import jax.numpy as jnp
from jax.experimental import pallas as pl
from jax.experimental.pallas import tpu as pltpu

_LEAKY = 0.1
_EPS = 1e-5
_B = 64
_KT = 7
_TK = 896


def _leaky(v):
    return jnp.maximum(v, _LEAKY * v)


def _mega_kernel(p1_ref, w1_ref, w2_ref, bn2g_ref, bn2b_ref,
                 fc1w_ref, fc1b_ref, g1_ref, be1_ref,
                 wh_ref, bh_ref, gq_ref, bq_ref, wq2_ref, bq2_ref,
                 d_ref, q_ref,
                 ph_ref, p2_ref, h2_ref, acc_ref, mv_ref):
    k = pl.program_id(0)

    @pl.when(k == 0)
    def _convs():
        zrow = jnp.zeros((2, 8, 64, 128), jnp.bfloat16)
        ph_ref[0, :, 0] = zrow
        ph_ref[1, :, 7] = zrow
        zcol = jnp.zeros((8, 64, 128), jnp.bfloat16)
        for hp in range(2):
            ph_ref[hp, 0, :, 0] = zcol
            ph_ref[hp, 1, :, 7] = zcol
        w1c = w1_ref[...].astype(jnp.bfloat16)
        y1 = jnp.dot(p1_ref[...], w1c, preferred_element_type=jnp.float32)
        y1 = _leaky(y1).astype(jnp.bfloat16)
        v = y1.reshape(7, 2, 7, 2, 64, 128)
        ph_ref[1, 1, 0:7, 0:7] = v[:, 0, :, 0]
        ph_ref[1, 0, 0:7, 1:8] = v[:, 0, :, 1]
        ph_ref[0, 1, 1:8, 0:7] = v[:, 1, :, 0]
        ph_ref[0, 0, 1:8, 1:8] = v[:, 1, :, 1]

        for i in range(4):
            for j in range(4):
                t = i * 4 + j
                tap = ph_ref[i % 2, j % 2,
                             i // 2:i // 2 + 7, j // 2:j // 2 + 7]
                p2_ref[:, t * 128:(t + 1) * 128] = tap.reshape(3136, 128)

        w2c = w2_ref[...].astype(jnp.bfloat16)
        y2 = jnp.dot(p2_ref[...], w2c, preferred_element_type=jnp.float32)
        h2_ref[...] = y2
        m = jnp.mean(y2, axis=0, keepdims=True)
        msq = jnp.mean(y2 * y2, axis=0, keepdims=True)
        var = msq - m * m
        s = jax.lax.rsqrt(var + _EPS) * bn2g_ref[...]
        mv_ref[0:1, :] = s
        mv_ref[1:2, :] = bn2b_ref[...] - m * s
        acc_ref[...] = jnp.zeros_like(acc_ref)

    bns = mv_ref[0:1, :]
    bnt = mv_ref[1:2, :]
    tot = None
    for s in range(7):
        row = pl.multiple_of((k * 7 + s) * 64, 64)
        lhs = _leaky(h2_ref[pl.ds(row, 64), :] * bns + bnt)
        d = jnp.dot(lhs, fc1w_ref[s * 128:(s + 1) * 128, :],
                    preferred_element_type=jnp.float32)
        tot = d if tot is None else tot + d
    acc_ref[...] += tot

    @pl.when(k == _KT - 1)
    def _tail():
        y = acc_ref[...] + fc1b_ref[...]
        mean = jnp.mean(y, axis=0, keepdims=True)
        var = jnp.mean((y - mean) ** 2, axis=0, keepdims=True)
        h = _leaky((y - mean) * jax.lax.rsqrt(var + _EPS)
                   * g1_ref[...] + be1_ref[...])
        hh = jnp.dot(h, wh_ref[...],
                     preferred_element_type=jnp.float32) + bh_ref[...]
        d = hh[:, :128]
        qv = hh[:, 128:]
        lane = jax.lax.broadcasted_iota(jnp.int32, d.shape, 1)
        d = jnp.where(lane < 2, d, -jnp.inf)
        mx = jnp.max(d, axis=-1, keepdims=True)
        e = jnp.exp(d - mx)
        sm = e / jnp.sum(e, axis=-1, keepdims=True)
        d_ref[...] = sm[:, :2]
        qm = jnp.mean(qv, axis=0, keepdims=True)
        qvar = jnp.mean((qv - qm) ** 2, axis=0, keepdims=True)
        qn = _leaky((qv - qm) * jax.lax.rsqrt(qvar + _EPS)
                    * gq_ref[...] + bq_ref[...])
        qo = jnp.dot(qn, wq2_ref[...],
                     preferred_element_type=jnp.float32) + bq2_ref[...]
        q_ref[...] = qo[:, :12]


def kernel(w1, w2, bn2_g, bn2_b, fc1_w, fc1_b, bnfc1_g, bnfc1_b,
           w_head, b_head, bnq1_g, bnq1_b, wq2, bq2, x):
    xt = jnp.transpose(x.reshape(_B, 28, 28), (1, 2, 0))
    xp = jnp.pad(xt, ((1, 1), (1, 1), (0, 0)))
    cols = [xp[i:i + 28:2, j:j + 28:2, :]
            for i in range(4) for j in range(4)]
    p1 = jnp.stack(cols, axis=-1).astype(jnp.bfloat16)
    p1 = p1.reshape(14 * 14 * _B, 16)

    cconst = lambda k: (0, 0)
    d_pad, q_pad = pl.pallas_call(
        _mega_kernel,
        out_shape=(jax.ShapeDtypeStruct((_B, 2), jnp.float32),
                   jax.ShapeDtypeStruct((_B, 12), jnp.float32)),
        grid=(_KT,),
        in_specs=[
            pl.BlockSpec((12544, 16), cconst),
            pl.BlockSpec((16, 128), cconst),
            pl.BlockSpec((2048, 128), cconst),
            pl.BlockSpec((1, 128), cconst),
            pl.BlockSpec((1, 128), cconst),
            pl.BlockSpec((_TK, 1024), lambda k: (k, 0)),
            pl.BlockSpec((1, 1024), cconst),
            pl.BlockSpec((1, 1024), cconst),
            pl.BlockSpec((1, 1024), cconst),
            pl.BlockSpec((1024, 256), cconst),
            pl.BlockSpec((1, 256), cconst),
            pl.BlockSpec((1, 128), cconst),
            pl.BlockSpec((1, 128), cconst),
            pl.BlockSpec((128, 128), cconst),
            pl.BlockSpec((1, 128), cconst),
        ],
        out_specs=(pl.BlockSpec((_B, 2), cconst),
                   pl.BlockSpec((_B, 12), cconst)),
        scratch_shapes=[
            pltpu.VMEM((2, 2, 8, 8, 64, 128), jnp.bfloat16),
            pltpu.VMEM((3136, 2048), jnp.bfloat16),
            pltpu.VMEM((3136, 128), jnp.float32),
            pltpu.VMEM((_B, 1024), jnp.float32),
            pltpu.VMEM((2, 128), jnp.float32),
        ],
        compiler_params=pltpu.CompilerParams(
            dimension_semantics=("arbitrary",),
            vmem_limit_bytes=56 * 1024 * 1024,
        ),
    )(p1, w1, w2, bn2_g, bn2_b, fc1_w, fc1_b, bnfc1_g, bnfc1_b,
      w_head, b_head, bnq1_g, bnq1_b, wq2, bq2)
    return d_pad, q_pad


# speedup vs baseline: 1.2426x; 1.2426x over previous
import jax
import jax.numpy as jnp
from jax.experimental import pallas as pl
from jax.experimental.pallas import tpu as pltpu

_LEAKY = 0.1
_EPS = 1e-5
_B = 64
_KT = 7
_TK = 896


def _leaky(v):
    return jnp.maximum(v, _LEAKY * v)


def _mega_kernel(p1_ref, w1_ref, w2_ref, bn2g_ref, bn2b_ref,
                 fc1w_ref, fc1b_ref, g1_ref, be1_ref,
                 wh_ref, bh_ref, gq_ref, bq_ref, wq2_ref, bq2_ref,
                 d_ref, q_ref,
                 ph_ref, p2_ref, h2_ref, acc_ref, mv_ref):
    k = pl.program_id(0)

    @pl.when(k == 0)
    def _convs():
        zrow = jnp.zeros((2, 8, 64, 128), jnp.bfloat16)
        ph_ref[0, :, 0] = zrow
        ph_ref[1, :, 7] = zrow
        zcol = jnp.zeros((8, 64, 128), jnp.bfloat16)
        for hp in range(2):
            ph_ref[hp, 0, :, 0] = zcol
            ph_ref[hp, 1, :, 7] = zcol
        w1c = w1_ref[...].astype(jnp.bfloat16)
        y1 = jnp.dot(p1_ref[...], w1c, preferred_element_type=jnp.float32)
        y1 = _leaky(y1).astype(jnp.bfloat16)
        v = y1.reshape(7, 2, 7, 2, 64, 128)
        ph_ref[1, 1, 0:7, 0:7] = v[:, 0, :, 0]
        ph_ref[1, 0, 0:7, 1:8] = v[:, 0, :, 1]
        ph_ref[0, 1, 1:8, 0:7] = v[:, 1, :, 0]
        ph_ref[0, 0, 1:8, 1:8] = v[:, 1, :, 1]

        for i in range(4):
            for j in range(4):
                t = i * 4 + j
                tap = ph_ref[i % 2, j % 2,
                             i // 2:i // 2 + 7, j // 2:j // 2 + 7]
                p2_ref[:, t * 128:(t + 1) * 128] = tap.reshape(3136, 128)

        w2c = w2_ref[...].astype(jnp.bfloat16)
        y2 = jnp.dot(p2_ref[...], w2c, preferred_element_type=jnp.float32)
        h2_ref[...] = y2
        m = jnp.mean(y2, axis=0, keepdims=True)
        msq = jnp.mean(y2 * y2, axis=0, keepdims=True)
        var = msq - m * m
        s = jax.lax.rsqrt(var + _EPS) * bn2g_ref[...]
        mv_ref[0:1, :] = s
        mv_ref[1:2, :] = bn2b_ref[...] - m * s
        acc_ref[...] = jnp.zeros_like(acc_ref)

    bns = mv_ref[0:1, :]
    bnt = mv_ref[1:2, :]
    tot = None
    for s in range(7):
        row = pl.multiple_of((k * 7 + s) * 64, 64)
        lhs = _leaky(h2_ref[pl.ds(row, 64), :] * bns + bnt)
        d = jnp.dot(lhs, fc1w_ref[s * 128:(s + 1) * 128, :],
                    preferred_element_type=jnp.float32)
        tot = d if tot is None else tot + d
    acc_ref[...] += tot

    @pl.when(k == _KT - 1)
    def _tail():
        y = acc_ref[...] + fc1b_ref[...]
        mean = jnp.mean(y, axis=0, keepdims=True)
        var = jnp.mean((y - mean) ** 2, axis=0, keepdims=True)
        h = _leaky((y - mean) * jax.lax.rsqrt(var + _EPS)
                   * g1_ref[...] + be1_ref[...])
        hh = jnp.dot(h, wh_ref[...],
                     preferred_element_type=jnp.float32) + bh_ref[...]
        d = hh[:, :128]
        qv = hh[:, 128:]
        lane = jax.lax.broadcasted_iota(jnp.int32, d.shape, 1)
        d = jnp.where(lane < 2, d, -jnp.inf)
        mx = jnp.max(d, axis=-1, keepdims=True)
        e = jnp.exp(d - mx)
        sm = e / jnp.sum(e, axis=-1, keepdims=True)
        d_ref[...] = sm[:, :2]
        qm = jnp.mean(qv, axis=0, keepdims=True)
        qvar = jnp.mean((qv - qm) ** 2, axis=0, keepdims=True)
        qn = _leaky((qv - qm) * jax.lax.rsqrt(qvar + _EPS)
                    * gq_ref[...] + bq_ref[...])
        qo = jnp.dot(qn, wq2_ref[...],
                     preferred_element_type=jnp.float32) + bq2_ref[...]
        q_ref[...] = qo[:, :12]


def kernel(w1, w2, bn2_g, bn2_b, fc1_w, fc1_b, bnfc1_g, bnfc1_b,
           w_head, b_head, bnq1_g, bnq1_b, wq2, bq2, x):
    xpb = jnp.pad(x.reshape(_B, 28, 28), ((0, 0), (1, 1), (1, 1)))
    a = jnp.concatenate(
        [xpb[:, 0:28, :].reshape(_B, 14, 2, 30),
         xpb[:, 2:30, :].reshape(_B, 14, 2, 30)], axis=2)
    c = jnp.concatenate(
        [a[..., 0:28].reshape(_B, 14, 4, 14, 2),
         a[..., 2:30].reshape(_B, 14, 4, 14, 2)], axis=4)
    p1 = c.transpose(1, 3, 0, 2, 4).astype(jnp.bfloat16)
    p1 = p1.reshape(14 * 14 * _B, 16)

    cconst = lambda k: (0, 0)
    d_pad, q_pad = pl.pallas_call(
        _mega_kernel,
        out_shape=(jax.ShapeDtypeStruct((_B, 2), jnp.float32),
                   jax.ShapeDtypeStruct((_B, 12), jnp.float32)),
        grid=(_KT,),
        in_specs=[
            pl.BlockSpec((12544, 16), cconst),
            pl.BlockSpec((16, 128), cconst),
            pl.BlockSpec((2048, 128), cconst),
            pl.BlockSpec((1, 128), cconst),
            pl.BlockSpec((1, 128), cconst),
            pl.BlockSpec((_TK, 1024), lambda k: (k, 0)),
            pl.BlockSpec((1, 1024), cconst),
            pl.BlockSpec((1, 1024), cconst),
            pl.BlockSpec((1, 1024), cconst),
            pl.BlockSpec((1024, 256), cconst),
            pl.BlockSpec((1, 256), cconst),
            pl.BlockSpec((1, 128), cconst),
            pl.BlockSpec((1, 128), cconst),
            pl.BlockSpec((128, 128), cconst),
            pl.BlockSpec((1, 128), cconst),
        ],
        out_specs=(pl.BlockSpec((_B, 2), cconst),
                   pl.BlockSpec((_B, 12), cconst)),
        scratch_shapes=[
            pltpu.VMEM((2, 2, 8, 8, 64, 128), jnp.bfloat16),
            pltpu.VMEM((3136, 2048), jnp.bfloat16),
            pltpu.VMEM((3136, 128), jnp.float32),
            pltpu.VMEM((_B, 1024), jnp.float32),
            pltpu.VMEM((2, 128), jnp.float32),
        ],
        compiler_params=pltpu.CompilerParams(
            dimension_semantics=("arbitrary",),
            vmem_limit_bytes=56 * 1024 * 1024,
        ),
    )(p1, w1, w2, bn2_g, bn2_b, fc1_w, fc1_b, bnfc1_g, bnfc1_b,
      w_head, b_head, bnq1_g, bnq1_b, wq2, bq2)
    return d_pad, q_pad
